# Initial kernel scaffold; baseline (speedup 1.0000x reference)
import jax
import jax.numpy as jnp
from jax.experimental import pallas as pl


def kernel(spikes):
    raise NotImplementedError("write your pallas kernel here")


# speedup vs baseline: 7.7634x; 7.7634x over previous
import functools

import jax
import jax.numpy as jnp
from jax import lax
from jax.experimental import pallas as pl
from jax.experimental.pallas import tpu as pltpu

_B, _L, _K = 64, 2048, 256
_THR = 0.5
_BIG = 1 << 30


def _wta_body(x_ref, idx_ref, w_ref, y_ref):
    x = x_ref[0]
    s = x > _THR
    ii = lax.broadcasted_iota(jnp.int32, (_L, _K), 0)
    kk = lax.broadcasted_iota(jnp.int32, (_L, _K), 1)
    flat = ii * _K + kk
    ff = jnp.min(jnp.where(s, flat, _BIG))
    has_any = ff < _BIG
    k_star = lax.rem(ff, _K)
    total = jnp.sum(x, axis=0, keepdims=True)
    kk1 = lax.broadcasted_iota(jnp.int32, (1, _K), 1)
    maxv = jnp.max(total)
    k_fb = jnp.min(jnp.where(total == maxv, kk1, _BIG))
    idx = jnp.where(has_any, k_star, k_fb)
    w = (kk1 == idx).astype(jnp.float32)
    idx_ref[0] = jnp.full((1, 1), idx, jnp.int32)
    w_ref[0] = w
    y_ref[0] = x * w


@jax.jit
def kernel(spikes):
    idx3, w3, y = pl.pallas_call(
        _wta_body,
        grid=(_B,),
        in_specs=[pl.BlockSpec((1, _L, _K), lambda b: (b, 0, 0))],
        out_specs=[
            pl.BlockSpec((1, 1, 1), lambda b: (b, 0, 0)),
            pl.BlockSpec((1, 1, _K), lambda b: (b, 0, 0)),
            pl.BlockSpec((1, _L, _K), lambda b: (b, 0, 0)),
        ],
        out_shape=[
            jax.ShapeDtypeStruct((_B, 1, 1), jnp.int32),
            jax.ShapeDtypeStruct((_B, 1, _K), jnp.float32),
            jax.ShapeDtypeStruct((_B, _L, _K), jnp.float32),
        ],
    )(spikes)
    return idx3[:, 0, 0], w3[:, 0, :], y


# speedup vs baseline: 9.9511x; 1.2818x over previous
import functools

import jax
import jax.numpy as jnp
from jax import lax
from jax.experimental import pallas as pl
from jax.experimental.pallas import tpu as pltpu

_B, _L, _K = 64, 2048, 256
_THR = 0.5
_BIG = 1 << 30


_PRE = 8


def _wta_body(x_ref, idx_ref, w_ref, y_ref, idx_s):
    x = x_ref[0]
    kk1 = lax.broadcasted_iota(jnp.int32, (1, _K), 1)
    xp = x[0:_PRE, :]
    iip = lax.broadcasted_iota(jnp.int32, (_PRE, _K), 0)
    kkp = lax.broadcasted_iota(jnp.int32, (_PRE, _K), 1)
    ffp = jnp.min(jnp.where(xp > _THR, iip * _K + kkp, _BIG))

    @pl.when(ffp < _BIG)
    def _():
        idx_s[0] = lax.rem(ffp, _K)

    @pl.when(ffp >= _BIG)
    def _():
        s = x > _THR
        ii = lax.broadcasted_iota(jnp.int32, (_L, _K), 0)
        kk = lax.broadcasted_iota(jnp.int32, (_L, _K), 1)
        ff = jnp.min(jnp.where(s, ii * _K + kk, _BIG))
        total = jnp.sum(x, axis=0, keepdims=True)
        maxv = jnp.max(total)
        k_fb = jnp.min(jnp.where(total == maxv, kk1, _BIG))
        idx_s[0] = jnp.where(ff < _BIG, lax.rem(ff, _K), k_fb)

    idx = idx_s[0]
    w = (kk1 == idx).astype(jnp.float32)
    idx_ref[0] = jnp.full((1, 1), idx, jnp.int32)
    w_ref[0] = w
    y_ref[0] = x * w


@jax.jit
def kernel(spikes):
    idx3, w3, y = pl.pallas_call(
        _wta_body,
        grid=(_B,),
        in_specs=[pl.BlockSpec((1, _L, _K), lambda b: (b, 0, 0))],
        out_specs=[
            pl.BlockSpec((1, 1, 1), lambda b: (b, 0, 0)),
            pl.BlockSpec((1, 1, _K), lambda b: (b, 0, 0)),
            pl.BlockSpec((1, _L, _K), lambda b: (b, 0, 0)),
        ],
        out_shape=[
            jax.ShapeDtypeStruct((_B, 1, 1), jnp.int32),
            jax.ShapeDtypeStruct((_B, 1, _K), jnp.float32),
            jax.ShapeDtypeStruct((_B, _L, _K), jnp.float32),
        ],
        scratch_shapes=[pltpu.SMEM((1,), jnp.int32)],
    )(spikes)
    return idx3[:, 0, 0], w3[:, 0, :], y
